# baseline (speedup 1.0000x reference)
import functools

import jax
import jax.numpy as jnp
from jax import lax
from jax.experimental import pallas as pl
from jax.experimental.pallas import tpu as pltpu
from jax.experimental.pallas import tpu_sc as plsc

B = 256
N0 = 1024
N1 = 512
BB = 8

NUM_CORES = 2
NUM_SUBCORES = 16
NUM_WORKERS = NUM_CORES * NUM_SUBCORES
ROWS_PER_WORKER = B // NUM_WORKERS
LANES = 16
CHUNKS = N1 // LANES


def _tc_body(w_ref, p0_ref, renorm_ref, ssq_ref):
    w = w_ref[...]
    s = jnp.sum(w, axis=1) * (1.0 / 12.0)
    m = jnp.max(s, axis=1)
    m1 = m[:, 1:]
    denom = jnp.sum(m1, axis=-1, keepdims=True)
    renorm = m1 / denom
    renorm_ref[...] = renorm
    d = p0_ref[...] - renorm
    part = jnp.sum(d * d)

    @pl.when(pl.program_id(0) == 0)
    def _():
        ssq_ref[0, 0] = 0.0

    ssq_ref[0, 0] += part


def _tc_renorm_loss0(w, p0):
    grid = B // BB
    return pl.pallas_call(
        _tc_body,
        grid=(grid,),
        in_specs=[
            pl.BlockSpec((BB, 12, 12, 1025), lambda i: (i, 0, 0, 0)),
            pl.BlockSpec((BB, N0), lambda i: (i, 0)),
        ],
        out_specs=[
            pl.BlockSpec((BB, N0), lambda i: (i, 0)),
            pl.BlockSpec(block_shape=(1, 1), index_map=lambda i: (0, 0),
                         memory_space=pltpu.SMEM),
        ],
        out_shape=[
            jax.ShapeDtypeStruct((B, N0), jnp.float32),
            jax.ShapeDtypeStruct((1, 1), jnp.float32),
        ],
    )(w, p0)


def _sc_body(renorm_hbm, idx_hbm, p1_hbm, out_hbm,
             row_v, idx_v, p1_v, g_v, acc_v):
    wid = lax.axis_index("s") * NUM_CORES + lax.axis_index("c")
    acc = jnp.zeros((LANES,), jnp.float32)
    for r in range(ROWS_PER_WORKER):
        b = wid * ROWS_PER_WORKER + r
        pltpu.sync_copy(renorm_hbm.at[pl.ds(b * N0, N0)], row_v)
        pltpu.sync_copy(idx_hbm.at[pl.ds(b * N1, N1)], idx_v)
        pltpu.sync_copy(p1_hbm.at[pl.ds(b * N1, N1)], p1_v)
        s = jnp.zeros((LANES,), jnp.float32)
        for j in range(CHUNKS):
            iv = idx_v[pl.ds(j * LANES, LANES)]
            g = plsc.load_gather(row_v, [iv])
            g_v[pl.ds(j * LANES, LANES)] = g
            s = s + g
        total_v = lax.broadcast(jnp.sum(s), (LANES,))
        inv_v = jnp.ones((LANES,), jnp.float32) / total_v
        for j in range(CHUNKS):
            d = p1_v[pl.ds(j * LANES, LANES)] - g_v[pl.ds(j * LANES, LANES)] * inv_v
            acc = acc + d * d
    acc_v[...] = acc
    pltpu.sync_copy(acc_v, out_hbm.at[pl.ds(wid * LANES, LANES)])


@functools.cache
def _sc_gather_loss1():
    return pl.kernel(
        _sc_body,
        mesh=plsc.VectorSubcoreMesh(core_axis_name="c", subcore_axis_name="s"),
        out_type=jax.ShapeDtypeStruct((NUM_WORKERS * LANES,), jnp.float32),
        scratch_types=[
            pltpu.VMEM((N0,), jnp.float32),
            pltpu.VMEM((N1,), jnp.int32),
            pltpu.VMEM((N1,), jnp.float32),
            pltpu.VMEM((N1,), jnp.float32),
            pltpu.VMEM((LANES,), jnp.float32),
        ],
        compiler_params=pltpu.CompilerParams(needs_layout_passes=False),
    )


def kernel(pred_logits_0, pred_logits_1, cls_attn_weights,
           kept_token_idx_0, kept_token_idx_1):
    renorm, ssq0 = _tc_renorm_loss0(cls_attn_weights, pred_logits_0)
    partials = _sc_gather_loss1()(renorm.reshape(-1),
                                  kept_token_idx_0.reshape(-1),
                                  pred_logits_1.reshape(-1))
    loss0 = 100.0 * ssq0[0, 0] / (B * N0)
    loss1 = 100.0 * jnp.sum(partials) / (B * N1)
    return loss0 + loss1


# speedup vs baseline: 2.4816x; 2.4816x over previous
import functools

import jax
import jax.numpy as jnp
from jax import lax
from jax.experimental import pallas as pl
from jax.experimental.pallas import tpu as pltpu
from jax.experimental.pallas import tpu_sc as plsc

B = 256
N0 = 1024
N1 = 512
NK = 1025
KB = 128
NKC = 9
NH = 12

NUM_CORES = 2
NUM_SUBCORES = 16
NUM_WORKERS = NUM_CORES * NUM_SUBCORES
ROWS_PER_WORKER = B // NUM_WORKERS
LANES = 16
CHUNKS = N1 // LANES


def _tc_body(w_ref, p0_ref, renorm_ref, ssq_ref, cls_ref, acc_ref):
    c = pl.program_id(0)
    j = pl.program_id(1)
    w = w_ref[...]
    m = jnp.sum(w, axis=(0, 1)) * (1.0 / 12.0)

    @pl.when(j == 0)
    def _():
        acc_ref[...] = m

    @pl.when(j != 0)
    def _():
        acc_ref[...] = jnp.maximum(acc_ref[...], m)

    @pl.when(j == NH - 1)
    def _():
        cls_ref[pl.ds(c * KB, KB), :] = acc_ref[...]

    @pl.when((c == NKC - 1) & (j == NH - 1))
    def _():
        m1 = cls_ref[pl.ds(1, N0), :]
        denom = jnp.sum(m1, axis=0, keepdims=True)
        renorm_kb = m1 / denom
        renorm = jnp.transpose(renorm_kb)
        renorm_ref[...] = renorm
        d = p0_ref[...] - renorm
        ssq_ref[0, 0] = jnp.sum(d * d)


def _tc_renorm_loss0(w4, p0):
    return pl.pallas_call(
        _tc_body,
        grid=(NKC, NH),
        in_specs=[
            pl.BlockSpec((NH, 1, KB, B), lambda c, j: (0, j, c, 0)),
            pl.BlockSpec((B, N0), lambda c, j: (0, 0)),
        ],
        out_specs=[
            pl.BlockSpec((B, N0), lambda c, j: (0, 0)),
            pl.BlockSpec(block_shape=(1, 1), index_map=lambda c, j: (0, 0),
                         memory_space=pltpu.SMEM),
        ],
        out_shape=[
            jax.ShapeDtypeStruct((B, N0), jnp.float32),
            jax.ShapeDtypeStruct((1, 1), jnp.float32),
        ],
        scratch_shapes=[
            pltpu.VMEM((NKC * KB, B), jnp.float32),
            pltpu.VMEM((KB, B), jnp.float32),
        ],
    )(w4, p0)


def _sc_body(renorm_hbm, idx_hbm, p1_hbm, out_hbm,
             row_v, idx_v, p1_v, g_v, acc_v):
    wid = lax.axis_index("s") * NUM_CORES + lax.axis_index("c")
    base = wid * ROWS_PER_WORKER
    pltpu.sync_copy(renorm_hbm.at[pl.ds(base * N0, ROWS_PER_WORKER * N0)], row_v)
    pltpu.sync_copy(idx_hbm.at[pl.ds(base * N1, ROWS_PER_WORKER * N1)], idx_v)
    pltpu.sync_copy(p1_hbm.at[pl.ds(base * N1, ROWS_PER_WORKER * N1)], p1_v)
    acc = jnp.zeros((LANES,), jnp.float32)
    for r in range(ROWS_PER_WORKER):
        s = jnp.zeros((LANES,), jnp.float32)
        for j in range(CHUNKS):
            iv = idx_v[pl.ds(r * N1 + j * LANES, LANES)] + jnp.int32(r * N0)
            g = plsc.load_gather(row_v, [iv])
            g_v[pl.ds(j * LANES, LANES)] = g
            s = s + g
        total_v = lax.broadcast(jnp.sum(s), (LANES,))
        inv_v = jnp.ones((LANES,), jnp.float32) / total_v
        for j in range(CHUNKS):
            d = (p1_v[pl.ds(r * N1 + j * LANES, LANES)]
                 - g_v[pl.ds(j * LANES, LANES)] * inv_v)
            acc = acc + d * d
    acc_v[...] = acc
    pltpu.sync_copy(acc_v, out_hbm.at[pl.ds(wid * LANES, LANES)])


@functools.cache
def _sc_gather_loss1():
    return pl.kernel(
        _sc_body,
        mesh=plsc.VectorSubcoreMesh(core_axis_name="c", subcore_axis_name="s"),
        out_type=jax.ShapeDtypeStruct((NUM_WORKERS * LANES,), jnp.float32),
        scratch_types=[
            pltpu.VMEM((ROWS_PER_WORKER * N0,), jnp.float32),
            pltpu.VMEM((ROWS_PER_WORKER * N1,), jnp.int32),
            pltpu.VMEM((ROWS_PER_WORKER * N1,), jnp.float32),
            pltpu.VMEM((N1,), jnp.float32),
            pltpu.VMEM((LANES,), jnp.float32),
        ],
        compiler_params=pltpu.CompilerParams(needs_layout_passes=False),
    )


def kernel(pred_logits_0, pred_logits_1, cls_attn_weights,
           kept_token_idx_0, kept_token_idx_1):
    w4 = jnp.transpose(cls_attn_weights, (1, 2, 3, 0))
    renorm, ssq0 = _tc_renorm_loss0(w4, pred_logits_0)
    partials = _sc_gather_loss1()(renorm.reshape(-1),
                                  kept_token_idx_0.reshape(-1),
                                  pred_logits_1.reshape(-1))
    loss0 = 100.0 * ssq0[0, 0] / (B * N0)
    loss1 = 100.0 * jnp.sum(partials) / (B * N1)
    return loss0 + loss1


# speedup vs baseline: 3.8159x; 1.5377x over previous
import functools

import jax
import jax.numpy as jnp
from jax import lax
from jax.experimental import pallas as pl
from jax.experimental.pallas import tpu as pltpu
from jax.experimental.pallas import tpu_sc as plsc

B = 256
N0 = 1024
N1 = 512
NK = 1025
KB = 128
NKC = 9
NH = 12

NUM_CORES = 2
NUM_SUBCORES = 16
NUM_WORKERS = NUM_CORES * NUM_SUBCORES
ROWS_PER_WORKER = B // NUM_WORKERS
LANES = 16
CHUNKS = N1 // LANES


def _tc_body(w_ref, p0_ref, renorm_ref, ssq_ref, cls_ref):
    j = pl.program_id(0)
    w = w_ref[...]
    m = jnp.sum(w, axis=(0, 1)) * (1.0 / 12.0)

    @pl.when(j == 0)
    def _():
        cls_ref[...] = m

    @pl.when(j != 0)
    def _():
        cls_ref[...] = jnp.maximum(cls_ref[...], m)

    @pl.when(j == NH - 1)
    def _():
        m1 = cls_ref[pl.ds(1, N0), :]
        denom = jnp.sum(m1, axis=0, keepdims=True)
        renorm_kb = m1 / denom
        renorm = jnp.transpose(renorm_kb)
        renorm_ref[...] = renorm
        d = p0_ref[...] - renorm
        ssq_ref[0, 0] = jnp.sum(d * d)


def _tc_renorm_loss0(w4, p0):
    return pl.pallas_call(
        _tc_body,
        grid=(NH,),
        in_specs=[
            pl.BlockSpec((NH, 1, NK, B), lambda j: (0, j, 0, 0)),
            pl.BlockSpec((B, N0), lambda j: (0, 0)),
        ],
        out_specs=[
            pl.BlockSpec((B, N0), lambda j: (0, 0)),
            pl.BlockSpec(block_shape=(1, 1), index_map=lambda j: (0, 0),
                         memory_space=pltpu.SMEM),
        ],
        out_shape=[
            jax.ShapeDtypeStruct((B, N0), jnp.float32),
            jax.ShapeDtypeStruct((1, 1), jnp.float32),
        ],
        scratch_shapes=[
            pltpu.VMEM((NK, B), jnp.float32),
        ],
    )(w4, p0)


def _sc_body(renorm_hbm, idx_hbm, p1_hbm, out_hbm,
             row_v, idx_v, p1_v, g_v, acc_v):
    wid = lax.axis_index("s") * NUM_CORES + lax.axis_index("c")
    base = wid * ROWS_PER_WORKER
    pltpu.sync_copy(renorm_hbm.at[pl.ds(base * N0, ROWS_PER_WORKER * N0)], row_v)
    pltpu.sync_copy(idx_hbm.at[pl.ds(base * N1, ROWS_PER_WORKER * N1)], idx_v)
    pltpu.sync_copy(p1_hbm.at[pl.ds(base * N1, ROWS_PER_WORKER * N1)], p1_v)
    acc = jnp.zeros((LANES,), jnp.float32)
    for r in range(ROWS_PER_WORKER):
        s = jnp.zeros((LANES,), jnp.float32)
        for j in range(CHUNKS):
            iv = idx_v[pl.ds(r * N1 + j * LANES, LANES)] + jnp.int32(r * N0)
            g = plsc.load_gather(row_v, [iv])
            g_v[pl.ds(j * LANES, LANES)] = g
            s = s + g
        total_v = lax.broadcast(jnp.sum(s), (LANES,))
        inv_v = jnp.ones((LANES,), jnp.float32) / total_v
        for j in range(CHUNKS):
            d = (p1_v[pl.ds(r * N1 + j * LANES, LANES)]
                 - g_v[pl.ds(j * LANES, LANES)] * inv_v)
            acc = acc + d * d
    acc_v[...] = acc
    pltpu.sync_copy(acc_v, out_hbm.at[pl.ds(wid * LANES, LANES)])


@functools.cache
def _sc_gather_loss1():
    return pl.kernel(
        _sc_body,
        mesh=plsc.VectorSubcoreMesh(core_axis_name="c", subcore_axis_name="s"),
        out_type=jax.ShapeDtypeStruct((NUM_WORKERS * LANES,), jnp.float32),
        scratch_types=[
            pltpu.VMEM((ROWS_PER_WORKER * N0,), jnp.float32),
            pltpu.VMEM((ROWS_PER_WORKER * N1,), jnp.int32),
            pltpu.VMEM((ROWS_PER_WORKER * N1,), jnp.float32),
            pltpu.VMEM((N1,), jnp.float32),
            pltpu.VMEM((LANES,), jnp.float32),
        ],
        compiler_params=pltpu.CompilerParams(needs_layout_passes=False),
    )


def kernel(pred_logits_0, pred_logits_1, cls_attn_weights,
           kept_token_idx_0, kept_token_idx_1):
    w4 = jnp.transpose(cls_attn_weights, (1, 2, 3, 0))
    renorm, ssq0 = _tc_renorm_loss0(w4, pred_logits_0)
    partials = _sc_gather_loss1()(renorm.reshape(-1),
                                  kept_token_idx_0.reshape(-1),
                                  pred_logits_1.reshape(-1))
    loss0 = 100.0 * ssq0[0, 0] / (B * N0)
    loss1 = 100.0 * jnp.sum(partials) / (B * N1)
    return loss0 + loss1
